# baseline (speedup 1.0000x reference)
import functools

import jax
import jax.numpy as jnp
from jax import lax
from jax.experimental import pallas as pl
from jax.experimental.pallas import tpu as pltpu
from jax.experimental.pallas import tpu_sc as plsc

N_NODES = 10000
D = 128
C_CLS = 40
E_EDGES = 320000

NC, NS = 2, 16
NW = NC * NS
CHUNK = 128
EPW = E_EDGES // NW
NCH = 80
EPW_PAD = NCH * CHUNK
NP = 10112
RPS = NP // NS

ROW_BLK = 1000

@functools.lru_cache(maxsize=None)
def _mesh():
    return plsc.VectorSubcoreMesh(core_axis_name="c", subcore_axis_name="s",
                                  num_cores=NC, num_subcores=NS)


def _worker_ids():
    c = lax.axis_index("c")
    s = lax.axis_index("s")
    return c, s, s * NC + c


def _deg_body(col_hbm, onehot_hbm, zeros_hbm, out_hbm, col_v, oh_v, acc):
    c, s, w = _worker_ids()
    pltpu.sync_copy(col_hbm.at[w], col_v)
    pltpu.sync_copy(onehot_hbm, oh_v)
    r0 = s * RPS
    pltpu.sync_copy(zeros_hbm.at[pl.ds(r0, RPS)], acc.at[pl.ds(r0, RPS)])
    plsc.subcore_barrier()

    def step(j, carry):
        pltpu.sync_copy(oh_v, acc.at[col_v.at[j]], add=True)
        return carry

    lax.fori_loop(0, NCH, step, 0)
    plsc.subcore_barrier()
    pltpu.sync_copy(acc.at[pl.ds(r0, RPS)], out_hbm.at[c].at[pl.ds(r0, RPS)])


@functools.lru_cache(maxsize=None)
def _sc_degree():
    return pl.kernel(
        _deg_body,
        out_type=jax.ShapeDtypeStruct((NC, NP, D), jnp.float32),
        mesh=_mesh(),
        scratch_types=[
            pltpu.VMEM((NCH, CHUNK), jnp.int32),
            pltpu.VMEM((CHUNK, D), jnp.float32),
            pltpu.VMEM_SHARED((NP, D), jnp.float32),
        ],
    )


def _spmm_body(row_hbm, col_hbm, h_hbm, zeros_hbm, out_hbm,
               row_v, col_v, gbuf, acc, sem):
    c, s, w = _worker_ids()
    pltpu.sync_copy(row_hbm.at[w], row_v)
    pltpu.sync_copy(col_hbm.at[w], col_v)
    r0 = s * RPS
    pltpu.sync_copy(zeros_hbm.at[pl.ds(r0, RPS)], acc.at[pl.ds(r0, RPS)])
    plsc.subcore_barrier()

    def step(j, carry):
        pltpu.async_copy(h_hbm.at[row_v.at[j]], gbuf, sem).wait()
        pltpu.sync_copy(gbuf, acc.at[col_v.at[j]], add=True)
        return carry

    lax.fori_loop(0, NCH, step, 0)
    plsc.subcore_barrier()
    pltpu.sync_copy(acc.at[pl.ds(r0, RPS)], out_hbm.at[c].at[pl.ds(r0, RPS)])


@functools.lru_cache(maxsize=None)
def _sc_spmm():
    return pl.kernel(
        _spmm_body,
        out_type=jax.ShapeDtypeStruct((NC, NP, D), jnp.float32),
        mesh=_mesh(),
        scratch_types=[
            pltpu.VMEM((NCH, CHUNK), jnp.int32),
            pltpu.VMEM((NCH, CHUNK), jnp.int32),
            pltpu.VMEM((CHUNK, D), jnp.float32),
            pltpu.VMEM_SHARED((NP, D), jnp.float32),
            pltpu.SemaphoreType.DMA,
        ],
    )


def _dinv_from(dp0, dp1):
    deg = dp0[:, 0:1] + dp1[:, 0:1]
    return jnp.where(deg > 0.0, lax.rsqrt(deg), 0.0)


def _enc_body(x_ref, w_ref, b_ref, dp_ref, h_ref, hs_ref):
    h = jnp.maximum(x_ref[...] @ w_ref[...] + b_ref[...], 0.0)
    dinv = _dinv_from(dp_ref[0], dp_ref[1])
    h_ref[...] = h
    hs_ref[...] = h * dinv


def _tc_encode(x, W_in, b_in, dp):
    grid = N_NODES // ROW_BLK
    return pl.pallas_call(
        _enc_body,
        grid=(grid,),
        in_specs=[
            pl.BlockSpec((ROW_BLK, D), lambda i: (i, 0)),
            pl.BlockSpec((D, D), lambda i: (0, 0)),
            pl.BlockSpec((1, D), lambda i: (0, 0)),
            pl.BlockSpec((NC, ROW_BLK, D), lambda i: (0, i, 0)),
        ],
        out_specs=[
            pl.BlockSpec((ROW_BLK, D), lambda i: (i, 0)),
            pl.BlockSpec((ROW_BLK, D), lambda i: (i, 0)),
        ],
        out_shape=[
            jax.ShapeDtypeStruct((N_NODES, D), jnp.float32),
            jax.ShapeDtypeStruct((N_NODES, D), jnp.float32),
        ],
    )(x, W_in, b_in, dp)


def _layer_body(sp_ref, dp_ref, h_ref, wa_ref, wb_ref, hn_ref, hs_ref):
    dinv = _dinv_from(dp_ref[0], dp_ref[1])
    h = h_ref[...]
    h_neigh = (sp_ref[0] + sp_ref[1]) * dinv
    out = h_neigh @ wa_ref[...] + h @ wb_ref[...] + h
    hn = jnp.maximum(out, 0.0)
    hn_ref[...] = hn
    hs_ref[...] = hn * dinv


def _tc_layer(sp, dp, h, W):
    grid = N_NODES // ROW_BLK
    return pl.pallas_call(
        _layer_body,
        grid=(grid,),
        in_specs=[
            pl.BlockSpec((NC, ROW_BLK, D), lambda i: (0, i, 0)),
            pl.BlockSpec((NC, ROW_BLK, D), lambda i: (0, i, 0)),
            pl.BlockSpec((ROW_BLK, D), lambda i: (i, 0)),
            pl.BlockSpec((D, D), lambda i: (0, 0)),
            pl.BlockSpec((D, D), lambda i: (0, 0)),
        ],
        out_specs=[
            pl.BlockSpec((ROW_BLK, D), lambda i: (i, 0)),
            pl.BlockSpec((ROW_BLK, D), lambda i: (i, 0)),
        ],
        out_shape=[
            jax.ShapeDtypeStruct((N_NODES, D), jnp.float32),
            jax.ShapeDtypeStruct((N_NODES, D), jnp.float32),
        ],
    )(sp, dp, h, W[:D], W[D:])


def _dec_body(h_ref, wca, bca, wsa, bsa, gcn, bcn, gsn, bsn,
              wcls, bcls, wscls, bscls, zc_ref, zs_ref, cl_ref, sl_ref):
    h = h_ref[...]

    def ln(t, g, b):
        m = jnp.mean(t, axis=-1, keepdims=True)
        v = jnp.mean((t - m) ** 2, axis=-1, keepdims=True)
        return (t - m) * lax.rsqrt(v + 1e-5) * g + b

    zc = ln(h + h @ wca[...] + bca[...], gcn[...], bcn[...])
    zs = ln(h + h @ wsa[...] + bsa[...], gsn[...], bsn[...])
    zc_ref[...] = zc
    zs_ref[...] = zs
    cl_ref[...] = zc @ wcls[...] + bcls[...]
    sl_ref[...] = zs @ wscls[...] + bscls[...]


def _tc_decompose(h, wca, bca, wsa, bsa, gcn, bcn, gsn, bsn,
                  wcls, bcls, wscls, bscls):
    grid = N_NODES // ROW_BLK
    full_dd = pl.BlockSpec((D, D), lambda i: (0, 0))
    full_1d = pl.BlockSpec((1, D), lambda i: (0, 0))
    full_dc = pl.BlockSpec((D, C_CLS), lambda i: (0, 0))
    full_1c = pl.BlockSpec((1, C_CLS), lambda i: (0, 0))
    return pl.pallas_call(
        _dec_body,
        grid=(grid,),
        in_specs=[
            pl.BlockSpec((ROW_BLK, D), lambda i: (i, 0)),
            full_dd, full_1d, full_dd, full_1d,
            full_1d, full_1d, full_1d, full_1d,
            full_dc, full_1c, full_dc, full_1c,
        ],
        out_specs=[
            pl.BlockSpec((ROW_BLK, D), lambda i: (i, 0)),
            pl.BlockSpec((ROW_BLK, D), lambda i: (i, 0)),
            pl.BlockSpec((ROW_BLK, C_CLS), lambda i: (i, 0)),
            pl.BlockSpec((ROW_BLK, C_CLS), lambda i: (i, 0)),
        ],
        out_shape=[
            jax.ShapeDtypeStruct((N_NODES, D), jnp.float32),
            jax.ShapeDtypeStruct((N_NODES, D), jnp.float32),
            jax.ShapeDtypeStruct((N_NODES, C_CLS), jnp.float32),
            jax.ShapeDtypeStruct((N_NODES, C_CLS), jnp.float32),
        ],
    )(h, wca, bca, wsa, bsa, gcn, bcn, gsn, bsn, wcls, bcls, wscls, bscls)


def kernel(x, edge_index, W_in, b_in, W_l1, W_l2, W_ca, b_ca, W_sa, b_sa,
           g_cn, b_cn, g_sn, b_sn, W_cls, b_cls, W_scls, b_scls):
    pad = EPW_PAD - EPW
    row = edge_index[0].reshape(NW, EPW)
    col = edge_index[1].reshape(NW, EPW)
    row = jnp.pad(row, ((0, 0), (0, pad))).reshape(NW, NCH, CHUNK)
    sink = N_NODES + (jnp.arange(pad, dtype=jnp.int32) % (NP - N_NODES))
    col = jnp.concatenate(
        [col, jnp.broadcast_to(sink, (NW, pad))], axis=1
    ).reshape(NW, NCH, CHUNK)
    onehot = jnp.zeros((CHUNK, D), jnp.float32).at[:, 0].set(1.0)
    zerosD = jnp.zeros((NP, D), jnp.float32)

    dp = _sc_degree()(col, onehot, zerosD)
    h0, hs0 = _tc_encode(x, W_in, b_in.reshape(1, D), dp)
    sp1 = _sc_spmm()(row, col, hs0, zerosD)
    h1, hs1 = _tc_layer(sp1, dp, h0, W_l1)
    sp2 = _sc_spmm()(row, col, hs1, zerosD)
    h2, _ = _tc_layer(sp2, dp, h1, W_l2)
    zc, zs, cl, sl = _tc_decompose(
        h2, W_ca, b_ca.reshape(1, D), W_sa, b_sa.reshape(1, D),
        g_cn.reshape(1, D), b_cn.reshape(1, D),
        g_sn.reshape(1, D), b_sn.reshape(1, D),
        W_cls, b_cls.reshape(1, C_CLS), W_scls, b_scls.reshape(1, C_CLS))
    return (zc, zs, cl, sl)


# speedup vs baseline: 1.0017x; 1.0017x over previous
import functools

import jax
import jax.numpy as jnp
from jax import lax
from jax.experimental import pallas as pl
from jax.experimental.pallas import tpu as pltpu
from jax.experimental.pallas import tpu_sc as plsc

N_NODES = 10000
D = 128
C_CLS = 40
E_EDGES = 320000

NC, NS = 2, 16
NW = NC * NS
CHUNK = 128
EPW = E_EDGES // NW
NCH = 80
EPW_PAD = NCH * CHUNK
NP = 10112
RPS = NP // NS

ROW_BLK = 1000

@functools.lru_cache(maxsize=None)
def _mesh():
    return plsc.VectorSubcoreMesh(core_axis_name="c", subcore_axis_name="s",
                                  num_cores=NC, num_subcores=NS)


def _worker_ids():
    c = lax.axis_index("c")
    s = lax.axis_index("s")
    return c, s, s * NC + c


def _deg_body(col_hbm, onehot_hbm, zeros_hbm, out_hbm, col_v, oh_v, acc):
    c, s, w = _worker_ids()
    pltpu.sync_copy(col_hbm.at[w], col_v)
    pltpu.sync_copy(onehot_hbm, oh_v)
    r0 = s * RPS
    pltpu.sync_copy(zeros_hbm.at[pl.ds(r0, RPS)], acc.at[pl.ds(r0, RPS)])
    plsc.subcore_barrier()

    def step(j, carry):
        pltpu.sync_copy(oh_v, acc.at[col_v.at[j]], add=True)
        return carry

    lax.fori_loop(0, NCH, step, 0)
    plsc.subcore_barrier()
    pltpu.sync_copy(acc.at[pl.ds(r0, RPS)], out_hbm.at[c].at[pl.ds(r0, RPS)])


@functools.lru_cache(maxsize=None)
def _sc_degree():
    return pl.kernel(
        _deg_body,
        out_type=jax.ShapeDtypeStruct((NC, NP, D), jnp.float32),
        mesh=_mesh(),
        scratch_types=[
            pltpu.VMEM((NCH, CHUNK), jnp.int32),
            pltpu.VMEM((CHUNK, D), jnp.float32),
            pltpu.VMEM_SHARED((NP, D), jnp.float32),
        ],
    )


def _spmm_body(row_hbm, col_hbm, h_hbm, zeros_hbm, out_hbm,
               row_v, col_v, gbuf, detune, acc, sem):
    c, s, w = _worker_ids()
    pltpu.sync_copy(row_hbm.at[w], row_v)
    pltpu.sync_copy(col_hbm.at[w], col_v)
    r0 = s * RPS
    pltpu.sync_copy(zeros_hbm.at[pl.ds(r0, RPS)], acc.at[pl.ds(r0, RPS)])
    plsc.subcore_barrier()

    def step(j, carry):
        pltpu.async_copy(h_hbm.at[row_v.at[j]], gbuf, sem).wait()
        pltpu.sync_copy(gbuf, acc.at[col_v.at[j]], add=True)
        return carry

    lax.fori_loop(0, NCH, step, 0)
    plsc.subcore_barrier()
    pltpu.sync_copy(acc.at[pl.ds(r0, RPS)], out_hbm.at[c].at[pl.ds(r0, RPS)])


@functools.lru_cache(maxsize=None)
def _sc_spmm():
    return pl.kernel(
        _spmm_body,
        out_type=jax.ShapeDtypeStruct((NC, NP, D), jnp.float32),
        mesh=_mesh(),
        scratch_types=[
            pltpu.VMEM((NCH, CHUNK), jnp.int32),
            pltpu.VMEM((NCH, CHUNK), jnp.int32),
            pltpu.VMEM((CHUNK, D), jnp.float32),
            pltpu.VMEM((40,), jnp.int32),
            pltpu.VMEM_SHARED((NP, D), jnp.float32),
            pltpu.SemaphoreType.DMA,
        ],
    )


def _dinv_from(dp0, dp1):
    deg = dp0[:, 0:1] + dp1[:, 0:1]
    return jnp.where(deg > 0.0, lax.rsqrt(deg), 0.0)


def _enc_body(x_ref, w_ref, b_ref, dp_ref, h_ref, hs_ref):
    h = jnp.maximum(x_ref[...] @ w_ref[...] + b_ref[...], 0.0)
    dinv = _dinv_from(dp_ref[0], dp_ref[1])
    h_ref[...] = h
    hs_ref[...] = h * dinv


def _tc_encode(x, W_in, b_in, dp):
    grid = N_NODES // ROW_BLK
    return pl.pallas_call(
        _enc_body,
        grid=(grid,),
        in_specs=[
            pl.BlockSpec((ROW_BLK, D), lambda i: (i, 0)),
            pl.BlockSpec((D, D), lambda i: (0, 0)),
            pl.BlockSpec((1, D), lambda i: (0, 0)),
            pl.BlockSpec((NC, ROW_BLK, D), lambda i: (0, i, 0)),
        ],
        out_specs=[
            pl.BlockSpec((ROW_BLK, D), lambda i: (i, 0)),
            pl.BlockSpec((ROW_BLK, D), lambda i: (i, 0)),
        ],
        out_shape=[
            jax.ShapeDtypeStruct((N_NODES, D), jnp.float32),
            jax.ShapeDtypeStruct((N_NODES, D), jnp.float32),
        ],
    )(x, W_in, b_in, dp)


def _layer_body(sp_ref, dp_ref, h_ref, wa_ref, wb_ref, hn_ref, hs_ref):
    dinv = _dinv_from(dp_ref[0], dp_ref[1])
    h = h_ref[...]
    h_neigh = (sp_ref[0] + sp_ref[1]) * dinv
    out = h_neigh @ wa_ref[...] + h @ wb_ref[...] + h
    hn = jnp.maximum(out, 0.0)
    hn_ref[...] = hn
    hs_ref[...] = hn * dinv


def _tc_layer(sp, dp, h, W):
    grid = N_NODES // ROW_BLK
    return pl.pallas_call(
        _layer_body,
        grid=(grid,),
        in_specs=[
            pl.BlockSpec((NC, ROW_BLK, D), lambda i: (0, i, 0)),
            pl.BlockSpec((NC, ROW_BLK, D), lambda i: (0, i, 0)),
            pl.BlockSpec((ROW_BLK, D), lambda i: (i, 0)),
            pl.BlockSpec((D, D), lambda i: (0, 0)),
            pl.BlockSpec((D, D), lambda i: (0, 0)),
        ],
        out_specs=[
            pl.BlockSpec((ROW_BLK, D), lambda i: (i, 0)),
            pl.BlockSpec((ROW_BLK, D), lambda i: (i, 0)),
        ],
        out_shape=[
            jax.ShapeDtypeStruct((N_NODES, D), jnp.float32),
            jax.ShapeDtypeStruct((N_NODES, D), jnp.float32),
        ],
    )(sp, dp, h, W[:D], W[D:])


def _dec_body(h_ref, wca, bca, wsa, bsa, gcn, bcn, gsn, bsn,
              wcls, bcls, wscls, bscls, zc_ref, zs_ref, cl_ref, sl_ref):
    h = h_ref[...]

    def ln(t, g, b):
        m = jnp.mean(t, axis=-1, keepdims=True)
        v = jnp.mean((t - m) ** 2, axis=-1, keepdims=True)
        return (t - m) * lax.rsqrt(v + 1e-5) * g + b

    zc = ln(h + h @ wca[...] + bca[...], gcn[...], bcn[...])
    zs = ln(h + h @ wsa[...] + bsa[...], gsn[...], bsn[...])
    zc_ref[...] = zc
    zs_ref[...] = zs
    cl_ref[...] = zc @ wcls[...] + bcls[...]
    sl_ref[...] = zs @ wscls[...] + bscls[...]


def _tc_decompose(h, wca, bca, wsa, bsa, gcn, bcn, gsn, bsn,
                  wcls, bcls, wscls, bscls):
    grid = N_NODES // ROW_BLK
    full_dd = pl.BlockSpec((D, D), lambda i: (0, 0))
    full_1d = pl.BlockSpec((1, D), lambda i: (0, 0))
    full_dc = pl.BlockSpec((D, C_CLS), lambda i: (0, 0))
    full_1c = pl.BlockSpec((1, C_CLS), lambda i: (0, 0))
    return pl.pallas_call(
        _dec_body,
        grid=(grid,),
        in_specs=[
            pl.BlockSpec((ROW_BLK, D), lambda i: (i, 0)),
            full_dd, full_1d, full_dd, full_1d,
            full_1d, full_1d, full_1d, full_1d,
            full_dc, full_1c, full_dc, full_1c,
        ],
        out_specs=[
            pl.BlockSpec((ROW_BLK, D), lambda i: (i, 0)),
            pl.BlockSpec((ROW_BLK, D), lambda i: (i, 0)),
            pl.BlockSpec((ROW_BLK, C_CLS), lambda i: (i, 0)),
            pl.BlockSpec((ROW_BLK, C_CLS), lambda i: (i, 0)),
        ],
        out_shape=[
            jax.ShapeDtypeStruct((N_NODES, D), jnp.float32),
            jax.ShapeDtypeStruct((N_NODES, D), jnp.float32),
            jax.ShapeDtypeStruct((N_NODES, C_CLS), jnp.float32),
            jax.ShapeDtypeStruct((N_NODES, C_CLS), jnp.float32),
        ],
    )(h, wca, bca, wsa, bsa, gcn, bcn, gsn, bsn, wcls, bcls, wscls, bscls)


def kernel(x, edge_index, W_in, b_in, W_l1, W_l2, W_ca, b_ca, W_sa, b_sa,
           g_cn, b_cn, g_sn, b_sn, W_cls, b_cls, W_scls, b_scls):
    pad = EPW_PAD - EPW
    row = edge_index[0].reshape(NW, EPW)
    col = edge_index[1].reshape(NW, EPW)
    row = jnp.pad(row, ((0, 0), (0, pad))).reshape(NW, NCH, CHUNK)
    sink = N_NODES + (jnp.arange(pad, dtype=jnp.int32) % (NP - N_NODES))
    col = jnp.concatenate(
        [col, jnp.broadcast_to(sink, (NW, pad))], axis=1
    ).reshape(NW, NCH, CHUNK)
    onehot = jnp.zeros((CHUNK, D), jnp.float32).at[:, 0].set(1.0)
    zerosD = jnp.zeros((NP, D), jnp.float32)

    dp = _sc_degree()(col, onehot, zerosD)
    h0, hs0 = _tc_encode(x, W_in, b_in.reshape(1, D), dp)
    sp1 = _sc_spmm()(row, col, hs0, zerosD)
    h1, hs1 = _tc_layer(sp1, dp, h0, W_l1)
    sp2 = _sc_spmm()(row, col, hs1, zerosD)
    h2, _ = _tc_layer(sp2, dp, h1, W_l2)
    zc, zs, cl, sl = _tc_decompose(
        h2, W_ca, b_ca.reshape(1, D), W_sa, b_sa.reshape(1, D),
        g_cn.reshape(1, D), b_cn.reshape(1, D),
        g_sn.reshape(1, D), b_sn.reshape(1, D),
        W_cls, b_cls.reshape(1, C_CLS), W_scls, b_scls.reshape(1, C_CLS))
    return (zc, zs, cl, sl)


# speedup vs baseline: 1.3976x; 1.3953x over previous
import functools

import jax
import jax.numpy as jnp
from jax import lax
from jax.experimental import pallas as pl
from jax.experimental.pallas import tpu as pltpu
from jax.experimental.pallas import tpu_sc as plsc

N_NODES = 10000
D = 128
C_CLS = 40
E_EDGES = 320000

NC, NS = 2, 16
NW = NC * NS
CHUNK = 128
EPW = E_EDGES // NW
NCH = 79
EPW_PAD = NCH * CHUNK
NP = 10112
RPS = NP // NS

ROW_BLK = 1000

@functools.lru_cache(maxsize=None)
def _mesh():
    return plsc.VectorSubcoreMesh(core_axis_name="c", subcore_axis_name="s",
                                  num_cores=NC, num_subcores=NS)


def _worker_ids():
    c = lax.axis_index("c")
    s = lax.axis_index("s")
    return c, s, s * NC + c


def _deg_body(col_hbm, onehot_hbm, zeros_hbm, out_hbm, col_v, oh_v, acc):
    c, s, w = _worker_ids()
    pltpu.sync_copy(col_hbm.at[w], col_v)
    pltpu.sync_copy(onehot_hbm, oh_v)
    r0 = s * RPS
    pltpu.sync_copy(zeros_hbm.at[pl.ds(r0, RPS)], acc.at[pl.ds(r0, RPS)])
    plsc.subcore_barrier()

    def step(j, carry):
        pltpu.sync_copy(oh_v, acc.at[col_v.at[j]], add=True)
        return carry

    lax.fori_loop(0, NCH, step, 0)
    plsc.subcore_barrier()
    pltpu.sync_copy(acc.at[pl.ds(r0, RPS)], out_hbm.at[c].at[pl.ds(r0, RPS)])


@functools.lru_cache(maxsize=None)
def _sc_degree():
    return pl.kernel(
        _deg_body,
        out_type=jax.ShapeDtypeStruct((NC, NP, D), jnp.float32),
        mesh=_mesh(),
        scratch_types=[
            pltpu.VMEM((NCH, CHUNK), jnp.int32),
            pltpu.VMEM((CHUNK, D), jnp.float32),
            pltpu.VMEM_SHARED((NP, D), jnp.float32),
        ],
    )


def _spmm_body(row_hbm, col_hbm, h_hbm, zeros_hbm, out_hbm,
               row_v, col_v, gbuf, detune, acc, sem):
    c, s, w = _worker_ids()
    pltpu.sync_copy(row_hbm.at[w], row_v)
    pltpu.sync_copy(col_hbm.at[w], col_v)
    pltpu.sync_copy(row_hbm.at[w].at[pl.ds(0, 2)], detune)
    r0 = s * RPS
    pltpu.sync_copy(zeros_hbm.at[pl.ds(r0, RPS)], acc.at[pl.ds(r0, RPS)])
    plsc.subcore_barrier()

    def step(j, carry):
        pltpu.async_copy(h_hbm.at[row_v.at[j]], gbuf, sem).wait()
        pltpu.sync_copy(gbuf, acc.at[col_v.at[j]], add=True)
        return carry

    lax.fori_loop(0, NCH, step, 0)
    plsc.subcore_barrier()
    pltpu.sync_copy(acc.at[pl.ds(r0, RPS)], out_hbm.at[c].at[pl.ds(r0, RPS)])


@functools.lru_cache(maxsize=None)
def _sc_spmm():
    return pl.kernel(
        _spmm_body,
        out_type=jax.ShapeDtypeStruct((NC, NP, D), jnp.float32),
        mesh=_mesh(),
        scratch_types=[
            pltpu.VMEM((NCH, CHUNK), jnp.int32),
            pltpu.VMEM((NCH, CHUNK), jnp.int32),
            pltpu.VMEM((CHUNK, D), jnp.float32),
            pltpu.VMEM((2, CHUNK), jnp.int32),
            pltpu.VMEM_SHARED((NP, D), jnp.float32),
            pltpu.SemaphoreType.DMA,
        ],
    )


def _dinv_from(dp0, dp1):
    deg = dp0[:, 0:1] + dp1[:, 0:1]
    return jnp.where(deg > 0.0, lax.rsqrt(deg), 0.0)


def _enc_body(x_ref, w_ref, b_ref, dp_ref, h_ref, hs_ref):
    h = jnp.maximum(x_ref[...] @ w_ref[...] + b_ref[...], 0.0)
    dinv = _dinv_from(dp_ref[0], dp_ref[1])
    h_ref[...] = h
    hs_ref[...] = h * dinv


def _tc_encode(x, W_in, b_in, dp):
    grid = N_NODES // ROW_BLK
    return pl.pallas_call(
        _enc_body,
        grid=(grid,),
        in_specs=[
            pl.BlockSpec((ROW_BLK, D), lambda i: (i, 0)),
            pl.BlockSpec((D, D), lambda i: (0, 0)),
            pl.BlockSpec((1, D), lambda i: (0, 0)),
            pl.BlockSpec((NC, ROW_BLK, D), lambda i: (0, i, 0)),
        ],
        out_specs=[
            pl.BlockSpec((ROW_BLK, D), lambda i: (i, 0)),
            pl.BlockSpec((ROW_BLK, D), lambda i: (i, 0)),
        ],
        out_shape=[
            jax.ShapeDtypeStruct((N_NODES, D), jnp.float32),
            jax.ShapeDtypeStruct((N_NODES, D), jnp.float32),
        ],
    )(x, W_in, b_in, dp)


def _layer_body(sp_ref, dp_ref, h_ref, wa_ref, wb_ref, hn_ref, hs_ref):
    dinv = _dinv_from(dp_ref[0], dp_ref[1])
    h = h_ref[...]
    h_neigh = (sp_ref[0] + sp_ref[1]) * dinv
    out = h_neigh @ wa_ref[...] + h @ wb_ref[...] + h
    hn = jnp.maximum(out, 0.0)
    hn_ref[...] = hn
    hs_ref[...] = hn * dinv


def _tc_layer(sp, dp, h, W):
    grid = N_NODES // ROW_BLK
    return pl.pallas_call(
        _layer_body,
        grid=(grid,),
        in_specs=[
            pl.BlockSpec((NC, ROW_BLK, D), lambda i: (0, i, 0)),
            pl.BlockSpec((NC, ROW_BLK, D), lambda i: (0, i, 0)),
            pl.BlockSpec((ROW_BLK, D), lambda i: (i, 0)),
            pl.BlockSpec((D, D), lambda i: (0, 0)),
            pl.BlockSpec((D, D), lambda i: (0, 0)),
        ],
        out_specs=[
            pl.BlockSpec((ROW_BLK, D), lambda i: (i, 0)),
            pl.BlockSpec((ROW_BLK, D), lambda i: (i, 0)),
        ],
        out_shape=[
            jax.ShapeDtypeStruct((N_NODES, D), jnp.float32),
            jax.ShapeDtypeStruct((N_NODES, D), jnp.float32),
        ],
    )(sp, dp, h, W[:D], W[D:])


def _dec_body(h_ref, wca, bca, wsa, bsa, gcn, bcn, gsn, bsn,
              wcls, bcls, wscls, bscls, zc_ref, zs_ref, cl_ref, sl_ref):
    h = h_ref[...]

    def ln(t, g, b):
        m = jnp.mean(t, axis=-1, keepdims=True)
        v = jnp.mean((t - m) ** 2, axis=-1, keepdims=True)
        return (t - m) * lax.rsqrt(v + 1e-5) * g + b

    zc = ln(h + h @ wca[...] + bca[...], gcn[...], bcn[...])
    zs = ln(h + h @ wsa[...] + bsa[...], gsn[...], bsn[...])
    zc_ref[...] = zc
    zs_ref[...] = zs
    cl_ref[...] = zc @ wcls[...] + bcls[...]
    sl_ref[...] = zs @ wscls[...] + bscls[...]


def _tc_decompose(h, wca, bca, wsa, bsa, gcn, bcn, gsn, bsn,
                  wcls, bcls, wscls, bscls):
    grid = N_NODES // ROW_BLK
    full_dd = pl.BlockSpec((D, D), lambda i: (0, 0))
    full_1d = pl.BlockSpec((1, D), lambda i: (0, 0))
    full_dc = pl.BlockSpec((D, C_CLS), lambda i: (0, 0))
    full_1c = pl.BlockSpec((1, C_CLS), lambda i: (0, 0))
    return pl.pallas_call(
        _dec_body,
        grid=(grid,),
        in_specs=[
            pl.BlockSpec((ROW_BLK, D), lambda i: (i, 0)),
            full_dd, full_1d, full_dd, full_1d,
            full_1d, full_1d, full_1d, full_1d,
            full_dc, full_1c, full_dc, full_1c,
        ],
        out_specs=[
            pl.BlockSpec((ROW_BLK, D), lambda i: (i, 0)),
            pl.BlockSpec((ROW_BLK, D), lambda i: (i, 0)),
            pl.BlockSpec((ROW_BLK, C_CLS), lambda i: (i, 0)),
            pl.BlockSpec((ROW_BLK, C_CLS), lambda i: (i, 0)),
        ],
        out_shape=[
            jax.ShapeDtypeStruct((N_NODES, D), jnp.float32),
            jax.ShapeDtypeStruct((N_NODES, D), jnp.float32),
            jax.ShapeDtypeStruct((N_NODES, C_CLS), jnp.float32),
            jax.ShapeDtypeStruct((N_NODES, C_CLS), jnp.float32),
        ],
    )(h, wca, bca, wsa, bsa, gcn, bcn, gsn, bsn, wcls, bcls, wscls, bscls)


def kernel(x, edge_index, W_in, b_in, W_l1, W_l2, W_ca, b_ca, W_sa, b_sa,
           g_cn, b_cn, g_sn, b_sn, W_cls, b_cls, W_scls, b_scls):
    pad = EPW_PAD - EPW
    row = edge_index[0].reshape(NW, EPW)
    col = edge_index[1].reshape(NW, EPW)
    row = jnp.pad(row, ((0, 0), (0, pad))).reshape(NW, NCH, CHUNK)
    sink = N_NODES + (jnp.arange(pad, dtype=jnp.int32) % (NP - N_NODES))
    col = jnp.concatenate(
        [col, jnp.broadcast_to(sink, (NW, pad))], axis=1
    ).reshape(NW, NCH, CHUNK)
    onehot = jnp.zeros((CHUNK, D), jnp.float32).at[:, 0].set(1.0)
    zerosD = jnp.zeros((NP, D), jnp.float32)

    dp = _sc_degree()(col, onehot, zerosD)
    h0, hs0 = _tc_encode(x, W_in, b_in.reshape(1, D), dp)
    sp1 = _sc_spmm()(row, col, hs0, zerosD)
    h1, hs1 = _tc_layer(sp1, dp, h0, W_l1)
    sp2 = _sc_spmm()(row, col, hs1, zerosD)
    h2, _ = _tc_layer(sp2, dp, h1, W_l2)
    zc, zs, cl, sl = _tc_decompose(
        h2, W_ca, b_ca.reshape(1, D), W_sa, b_sa.reshape(1, D),
        g_cn.reshape(1, D), b_cn.reshape(1, D),
        g_sn.reshape(1, D), b_sn.reshape(1, D),
        W_cls, b_cls.reshape(1, C_CLS), W_scls, b_scls.reshape(1, C_CLS))
    return (zc, zs, cl, sl)


# speedup vs baseline: 1.4861x; 1.0633x over previous
import functools

import jax
import jax.numpy as jnp
from jax import lax
from jax.experimental import pallas as pl
from jax.experimental.pallas import tpu as pltpu
from jax.experimental.pallas import tpu_sc as plsc

N_NODES = 10000
D = 128
C_CLS = 40
E_EDGES = 320000

NC, NS = 2, 16
NW = NC * NS
CHUNK = 128
EPW = E_EDGES // NW
NCH = 79
EPW_PAD = NCH * CHUNK
NP = 10112
RPS = NP // NS

ROW_BLK = 1000

@functools.lru_cache(maxsize=None)
def _mesh():
    return plsc.VectorSubcoreMesh(core_axis_name="c", subcore_axis_name="s",
                                  num_cores=NC, num_subcores=NS)


def _worker_ids():
    c = lax.axis_index("c")
    s = lax.axis_index("s")
    return c, s, s * NC + c


def _deg_body(col_hbm, onehot_hbm, zeros_hbm, out_hbm, col_v, oh_v, acc):
    c, s, w = _worker_ids()
    pltpu.sync_copy(col_hbm.at[w], col_v)
    pltpu.sync_copy(onehot_hbm, oh_v)
    r0 = s * RPS
    pltpu.sync_copy(zeros_hbm.at[pl.ds(r0, RPS)], acc.at[pl.ds(r0, RPS)])
    plsc.subcore_barrier()

    def step(j, carry):
        pltpu.sync_copy(oh_v, acc.at[col_v.at[j]], add=True)
        return carry

    lax.fori_loop(0, NCH, step, 0)
    plsc.subcore_barrier()
    pltpu.sync_copy(acc.at[pl.ds(r0, RPS)], out_hbm.at[c].at[pl.ds(r0, RPS)])


@functools.lru_cache(maxsize=None)
def _sc_degree():
    return pl.kernel(
        _deg_body,
        out_type=jax.ShapeDtypeStruct((NC, NP, D), jnp.float32),
        mesh=_mesh(),
        scratch_types=[
            pltpu.VMEM((NCH, CHUNK), jnp.int32),
            pltpu.VMEM((CHUNK, D), jnp.float32),
            pltpu.VMEM_SHARED((NP, D), jnp.float32),
        ],
    )


def _spmm_pair(h_hbm, acc, stg, gbuf, sems, t0):
    a0 = pltpu.async_copy(h_hbm.at[stg.at[t0].at[0]], gbuf.at[0], sems[0])
    a1 = pltpu.async_copy(h_hbm.at[stg.at[t0 + 1].at[0]], gbuf.at[1], sems[1])
    a0.wait()
    pltpu.sync_copy(gbuf.at[0], acc.at[stg.at[t0].at[1]], add=True)
    a1.wait()
    pltpu.sync_copy(gbuf.at[1], acc.at[stg.at[t0 + 1].at[1]], add=True)


def _spmm_body(rc_hbm, h_hbm, zeros_hbm, out_hbm, stg, gbuf, acc, sems):
    c, s, w = _worker_ids()
    r0 = s * RPS
    pltpu.sync_copy(zeros_hbm.at[pl.ds(r0, RPS)], acc.at[pl.ds(r0, RPS)])
    plsc.subcore_barrier()

    pltpu.sync_copy(rc_hbm.at[w].at[pl.ds(0, 40)], stg)

    def step0(i, carry):
        _spmm_pair(h_hbm, acc, stg, gbuf, sems, 2 * i)
        return carry

    lax.fori_loop(0, 20, step0, 0)
    pltpu.sync_copy(rc_hbm.at[w].at[pl.ds(40, 39)], stg.at[pl.ds(0, 39)])

    def step1(i, carry):
        _spmm_pair(h_hbm, acc, stg, gbuf, sems, 2 * i)
        return carry

    lax.fori_loop(0, 19, step1, 0)
    pltpu.async_copy(h_hbm.at[stg.at[38].at[0]], gbuf.at[0], sems[0]).wait()
    pltpu.sync_copy(gbuf.at[0], acc.at[stg.at[38].at[1]], add=True)

    plsc.subcore_barrier()
    pltpu.sync_copy(acc.at[pl.ds(r0, RPS)], out_hbm.at[c].at[pl.ds(r0, RPS)])


@functools.lru_cache(maxsize=None)
def _sc_spmm():
    return pl.kernel(
        _spmm_body,
        out_type=jax.ShapeDtypeStruct((NC, NP, D), jnp.float32),
        mesh=_mesh(),
        scratch_types=[
            pltpu.VMEM((40, 2, CHUNK), jnp.int32),
            pltpu.VMEM((2, CHUNK, D), jnp.float32),
            pltpu.VMEM_SHARED((NP, D), jnp.float32),
            [pltpu.SemaphoreType.DMA] * 2,
        ],
    )


def _dinv_from(dp0, dp1):
    deg = dp0[:, 0:1] + dp1[:, 0:1]
    return jnp.where(deg > 0.0, lax.rsqrt(deg), 0.0)


def _enc_body(x_ref, w_ref, b_ref, dp_ref, h_ref, hs_ref):
    h = jnp.maximum(x_ref[...] @ w_ref[...] + b_ref[...], 0.0)
    dinv = _dinv_from(dp_ref[0], dp_ref[1])
    h_ref[...] = h
    hs_ref[...] = h * dinv


def _tc_encode(x, W_in, b_in, dp):
    grid = N_NODES // ROW_BLK
    return pl.pallas_call(
        _enc_body,
        grid=(grid,),
        in_specs=[
            pl.BlockSpec((ROW_BLK, D), lambda i: (i, 0)),
            pl.BlockSpec((D, D), lambda i: (0, 0)),
            pl.BlockSpec((1, D), lambda i: (0, 0)),
            pl.BlockSpec((NC, ROW_BLK, D), lambda i: (0, i, 0)),
        ],
        out_specs=[
            pl.BlockSpec((ROW_BLK, D), lambda i: (i, 0)),
            pl.BlockSpec((ROW_BLK, D), lambda i: (i, 0)),
        ],
        out_shape=[
            jax.ShapeDtypeStruct((N_NODES, D), jnp.float32),
            jax.ShapeDtypeStruct((N_NODES, D), jnp.float32),
        ],
    )(x, W_in, b_in, dp)


def _layer_body(sp_ref, dp_ref, h_ref, wa_ref, wb_ref, hn_ref, hs_ref):
    dinv = _dinv_from(dp_ref[0], dp_ref[1])
    h = h_ref[...]
    h_neigh = (sp_ref[0] + sp_ref[1]) * dinv
    out = h_neigh @ wa_ref[...] + h @ wb_ref[...] + h
    hn = jnp.maximum(out, 0.0)
    hn_ref[...] = hn
    hs_ref[...] = hn * dinv


def _tc_layer(sp, dp, h, W):
    grid = N_NODES // ROW_BLK
    return pl.pallas_call(
        _layer_body,
        grid=(grid,),
        in_specs=[
            pl.BlockSpec((NC, ROW_BLK, D), lambda i: (0, i, 0)),
            pl.BlockSpec((NC, ROW_BLK, D), lambda i: (0, i, 0)),
            pl.BlockSpec((ROW_BLK, D), lambda i: (i, 0)),
            pl.BlockSpec((D, D), lambda i: (0, 0)),
            pl.BlockSpec((D, D), lambda i: (0, 0)),
        ],
        out_specs=[
            pl.BlockSpec((ROW_BLK, D), lambda i: (i, 0)),
            pl.BlockSpec((ROW_BLK, D), lambda i: (i, 0)),
        ],
        out_shape=[
            jax.ShapeDtypeStruct((N_NODES, D), jnp.float32),
            jax.ShapeDtypeStruct((N_NODES, D), jnp.float32),
        ],
    )(sp, dp, h, W[:D], W[D:])


def _dec_body(h_ref, wca, bca, wsa, bsa, gcn, bcn, gsn, bsn,
              wcls, bcls, wscls, bscls, zc_ref, zs_ref, cl_ref, sl_ref):
    h = h_ref[...]

    def ln(t, g, b):
        m = jnp.mean(t, axis=-1, keepdims=True)
        v = jnp.mean((t - m) ** 2, axis=-1, keepdims=True)
        return (t - m) * lax.rsqrt(v + 1e-5) * g + b

    zc = ln(h + h @ wca[...] + bca[...], gcn[...], bcn[...])
    zs = ln(h + h @ wsa[...] + bsa[...], gsn[...], bsn[...])
    zc_ref[...] = zc
    zs_ref[...] = zs
    cl_ref[...] = zc @ wcls[...] + bcls[...]
    sl_ref[...] = zs @ wscls[...] + bscls[...]


def _tc_decompose(h, wca, bca, wsa, bsa, gcn, bcn, gsn, bsn,
                  wcls, bcls, wscls, bscls):
    grid = N_NODES // ROW_BLK
    full_dd = pl.BlockSpec((D, D), lambda i: (0, 0))
    full_1d = pl.BlockSpec((1, D), lambda i: (0, 0))
    full_dc = pl.BlockSpec((D, C_CLS), lambda i: (0, 0))
    full_1c = pl.BlockSpec((1, C_CLS), lambda i: (0, 0))
    return pl.pallas_call(
        _dec_body,
        grid=(grid,),
        in_specs=[
            pl.BlockSpec((ROW_BLK, D), lambda i: (i, 0)),
            full_dd, full_1d, full_dd, full_1d,
            full_1d, full_1d, full_1d, full_1d,
            full_dc, full_1c, full_dc, full_1c,
        ],
        out_specs=[
            pl.BlockSpec((ROW_BLK, D), lambda i: (i, 0)),
            pl.BlockSpec((ROW_BLK, D), lambda i: (i, 0)),
            pl.BlockSpec((ROW_BLK, C_CLS), lambda i: (i, 0)),
            pl.BlockSpec((ROW_BLK, C_CLS), lambda i: (i, 0)),
        ],
        out_shape=[
            jax.ShapeDtypeStruct((N_NODES, D), jnp.float32),
            jax.ShapeDtypeStruct((N_NODES, D), jnp.float32),
            jax.ShapeDtypeStruct((N_NODES, C_CLS), jnp.float32),
            jax.ShapeDtypeStruct((N_NODES, C_CLS), jnp.float32),
        ],
    )(h, wca, bca, wsa, bsa, gcn, bcn, gsn, bsn, wcls, bcls, wscls, bscls)


def kernel(x, edge_index, W_in, b_in, W_l1, W_l2, W_ca, b_ca, W_sa, b_sa,
           g_cn, b_cn, g_sn, b_sn, W_cls, b_cls, W_scls, b_scls):
    pad = EPW_PAD - EPW
    row = edge_index[0].reshape(NW, EPW)
    col = edge_index[1].reshape(NW, EPW)
    row = jnp.pad(row, ((0, 0), (0, pad))).reshape(NW, NCH, CHUNK)
    sink = N_NODES + (jnp.arange(pad, dtype=jnp.int32) % (NP - N_NODES))
    col = jnp.concatenate(
        [col, jnp.broadcast_to(sink, (NW, pad))], axis=1
    ).reshape(NW, NCH, CHUNK)
    onehot = jnp.zeros((CHUNK, D), jnp.float32).at[:, 0].set(1.0)
    zerosD = jnp.zeros((NP, D), jnp.float32)

    rc = jnp.stack([row, col], axis=2)
    dp = _sc_degree()(col, onehot, zerosD)
    h0, hs0 = _tc_encode(x, W_in, b_in.reshape(1, D), dp)
    sp1 = _sc_spmm()(rc, hs0, zerosD)
    h1, hs1 = _tc_layer(sp1, dp, h0, W_l1)
    sp2 = _sc_spmm()(rc, hs1, zerosD)
    h2, _ = _tc_layer(sp2, dp, h1, W_l2)
    zc, zs, cl, sl = _tc_decompose(
        h2, W_ca, b_ca.reshape(1, D), W_sa, b_sa.reshape(1, D),
        g_cn.reshape(1, D), b_cn.reshape(1, D),
        g_sn.reshape(1, D), b_sn.reshape(1, D),
        W_cls, b_cls.reshape(1, C_CLS), W_scls, b_scls.reshape(1, C_CLS))
    return (zc, zs, cl, sl)
